# baseline (speedup 1.0000x reference)
import math

import jax
import jax.numpy as jnp
from jax.experimental import pallas as pl

_K = 4
_LOG_F = math.log(128.0)


def _ln_rows(x, g, b, eps=1e-5):
    m = jnp.mean(x, axis=-1, keepdims=True)
    v = jnp.mean((x - m) ** 2, axis=-1, keepdims=True)
    return (x - m) * jax.lax.rsqrt(v + eps) * g + b


def _gelu(x):
    return 0.5 * x * (1.0 + jax.lax.erf(x * (1.0 / math.sqrt(2.0))))


def _fused_kernel(x_ref, ln1g_ref, ln1b_ref, noopw_ref, noopb_ref,
                  cw1_ref, cb1_ref, cw2_ref, cb2_ref, cw3_ref, cb3_ref,
                  fw1a_ref, emb_ref, ln2g_ref, ln2b_ref,
                  fw1b_ref, fb1_ref, fw2_ref, fb2_ref, fw3_ref, fb3_ref,
                  out_ref):
    f32 = jnp.float32
    bf16 = jnp.bfloat16
    x = x_ref[...]
    bt = x.shape[0]
    ntot = cw3_ref.shape[1]
    ftot = fw3_ref.shape[1]

    xln = _ln_rows(x, ln1g_ref[...], ln1b_ref[...])
    noop = (jnp.sum(xln * noopw_ref[...], axis=-1, keepdims=True)
            + noopb_ref[...])
    h = _gelu(jnp.dot(xln, cw1_ref[...], preferred_element_type=f32)
              + cb1_ref[...])
    h = _gelu(jnp.dot(h, cw2_ref[...], preferred_element_type=f32)
              + cb2_ref[...])
    coarse = (jnp.dot(h, cw3_ref[...], preferred_element_type=f32)
              + cb3_ref[...])

    iota_n = jax.lax.broadcasted_iota(jnp.int32, (bt, ntot), 1)
    vals = coarse
    idxs = []
    for k in range(_K):
        m = jnp.max(vals, axis=-1, keepdims=True)
        idxk = jnp.min(jnp.where(vals == m, iota_n, ntot), axis=-1,
                       keepdims=True)
        idxs.append(idxk)
        vals = jnp.where(iota_n == idxk, jnp.float32(-jnp.inf), vals)

    t = jnp.dot(xln.astype(bf16), fw1a_ref[...], preferred_element_type=f32)
    upds = []
    for k in range(_K):
        oh = (iota_n == idxs[k]).astype(bf16)
        e = jnp.dot(oh, emb_ref[...], preferred_element_type=f32)
        e = _ln_rows(e, ln2g_ref[...], ln2b_ref[...])
        hf = _gelu(t + jnp.dot(e.astype(bf16), fw1b_ref[...],
                               preferred_element_type=f32) + fb1_ref[...])
        hf = _gelu(jnp.dot(hf.astype(bf16), fw2_ref[...],
                           preferred_element_type=f32) + fb2_ref[...])
        f = (jnp.dot(hf.astype(bf16), fw3_ref[...],
                     preferred_element_type=f32) + fb3_ref[...])
        m = jnp.max(f, axis=-1, keepdims=True)
        lse = m + jnp.log(jnp.sum(jnp.exp(f - m), axis=-1, keepdims=True))
        upds.append(f + _LOG_F - lse)

    jj = jax.lax.broadcasted_iota(jnp.int32, (ftot, 2048), 0)
    mm = jax.lax.broadcasted_iota(jnp.int32, (ftot, 2048), 1)
    T = (jj == 16 * (mm // 256) + mm % 16).astype(f32)
    c16 = jax.lax.broadcasted_iota(jnp.int32, (16, 2048), 0)
    m16 = jax.lax.broadcasted_iota(jnp.int32, (16, 2048), 1)
    M16 = (c16 == (m16 // 16) % 16).astype(f32)
    M16h = (c16 == (m16 // 16) % 16).astype(bf16)

    updbig = [jnp.dot(upds[k], T, preferred_element_type=f32)
              for k in range(_K)]
    sstack = jnp.concatenate(
        [(iota_n == idxs[k]).astype(bf16) for k in range(_K)], axis=0)

    pieces = [noop]
    for ch in range(16):
        seg = jnp.dot(coarse[:, 16 * ch:16 * (ch + 1)], M16,
                      preferred_element_type=f32) - _LOG_F
        sexp = jnp.dot(sstack[:, 16 * ch:16 * (ch + 1)], M16h,
                       preferred_element_type=f32)
        for k in range(_K):
            seg = seg + sexp[k * bt:(k + 1) * bt] * updbig[k]
        pieces.append(seg)
    out_ref[...] = jnp.concatenate(pieces, axis=-1)


def _full(w):
    return pl.BlockSpec(w.shape, lambda i: (0,) * w.ndim)


def kernel(x, ln1_g, ln1_b, noop_W, noop_b, cW1, cb1, cW2, cb2, cW3, cb3,
           emb, ln2_g, ln2_b, fW1, fb1, fW2, fb2, fW3, fb3):
    B, C = x.shape
    NTOT = cW3.shape[1]
    FTOT = fW3.shape[1]
    f32 = jnp.float32
    bf16 = jnp.bfloat16

    def row(v):
        return v.reshape(1, -1)

    fW1a = fW1[:C].astype(bf16)
    fW1b = fW1[C:].astype(bf16)
    fW2h = fW2.astype(bf16)
    fW3h = fW3.astype(bf16)
    embh = emb.astype(bf16)

    bt = 64
    ins = (x, row(ln1_g), row(ln1_b), row(noop_W), row(noop_b),
           cW1, row(cb1), cW2, row(cb2), cW3, row(cb3),
           fW1a, embh, row(ln2_g), row(ln2_b),
           fW1b, row(fb1), fW2h, row(fb2), fW3h, row(fb3))
    out = pl.pallas_call(
        _fused_kernel,
        grid=(B // bt,),
        in_specs=[pl.BlockSpec((bt, C), lambda i: (i, 0))]
        + [_full(v) for v in ins[1:]],
        out_specs=pl.BlockSpec((bt, 1 + NTOT * FTOT), lambda i: (i, 0)),
        out_shape=jax.ShapeDtypeStruct((B, 1 + NTOT * FTOT), f32),
    )(*ins)
    return out


# speedup vs baseline: 1.2010x; 1.2010x over previous
import math

import jax
import jax.numpy as jnp
from jax.experimental import pallas as pl

_K = 4
_LOG_F = math.log(128.0)


def _ln_rows(x, g, b, eps=1e-5):
    m = jnp.mean(x, axis=-1, keepdims=True)
    v = jnp.mean((x - m) ** 2, axis=-1, keepdims=True)
    return (x - m) * jax.lax.rsqrt(v + eps) * g + b


def _gelu(x):
    return 0.5 * x * (1.0 + jax.lax.erf(x * (1.0 / math.sqrt(2.0))))


def _coarse_kernel(x_ref, ln1g_ref, ln1b_ref, noopw_ref, noopb_ref,
                   cw1_ref, cb1_ref, cw2_ref, cb2_ref, cw3_ref, cb3_ref,
                   fw1a_ref,
                   noop_ref, t_ref, coarse_ref, idx_ref):
    x = x_ref[...]
    xln = _ln_rows(x, ln1g_ref[...], ln1b_ref[...])
    noop_ref[...] = (jnp.sum(xln * noopw_ref[...], axis=-1, keepdims=True)
                     + noopb_ref[...])
    h = _gelu(jnp.dot(xln, cw1_ref[...], preferred_element_type=jnp.float32)
              + cb1_ref[...])
    h = _gelu(jnp.dot(h, cw2_ref[...], preferred_element_type=jnp.float32)
              + cb2_ref[...])
    coarse = (jnp.dot(h, cw3_ref[...], preferred_element_type=jnp.float32)
              + cb3_ref[...])
    coarse_ref[...] = coarse
    t_ref[...] = jnp.dot(xln.astype(jnp.bfloat16), fw1a_ref[...],
                         preferred_element_type=jnp.float32)

    n = coarse.shape[-1]
    iota = jax.lax.broadcasted_iota(jnp.int32, coarse.shape, 1)
    vals = coarse
    for k in range(_K):
        m = jnp.max(vals, axis=-1, keepdims=True)
        idxk = jnp.min(jnp.where(vals == m, iota, n), axis=-1, keepdims=True)
        idx_ref[:, k:k + 1] = idxk
        vals = jnp.where(iota == idxk, jnp.float32(-jnp.inf), vals)


def _fine_kernel(t_ref, idx_ref, emb_ref, ln2g_ref, ln2b_ref,
                 fw1b_ref, fb1_ref, fw2_ref, fb2_ref, fw3_ref, fb3_ref,
                 upd_ref):
    t = t_ref[...]
    bt = t.shape[0]
    ntot = emb_ref.shape[0]
    ftot = fw3_ref.shape[1]
    iota_n = jax.lax.broadcasted_iota(jnp.int32, (bt, ntot), 1)
    for k in range(_K):
        idxk = idx_ref[:, k:k + 1]
        oh = (iota_n == idxk).astype(jnp.bfloat16)
        e = jnp.dot(oh, emb_ref[...], preferred_element_type=jnp.float32)
        e = _ln_rows(e, ln2g_ref[...], ln2b_ref[...])
        h = _gelu(t + jnp.dot(e.astype(jnp.bfloat16), fw1b_ref[...],
                              preferred_element_type=jnp.float32)
                  + fb1_ref[...])
        h = _gelu(jnp.dot(h.astype(jnp.bfloat16), fw2_ref[...],
                          preferred_element_type=jnp.float32)
                  + fb2_ref[...])
        f = (jnp.dot(h.astype(jnp.bfloat16), fw3_ref[...],
                     preferred_element_type=jnp.float32)
             + fb3_ref[...])
        m = jnp.max(f, axis=-1, keepdims=True)
        lse = m + jnp.log(jnp.sum(jnp.exp(f - m), axis=-1, keepdims=True))
        upd_ref[:, k * ftot:(k + 1) * ftot] = f + _LOG_F - lse


def _expand_kernel(noop_ref, coarse_ref, idx_ref, upd_ref, out_ref):
    bt = coarse_ref.shape[0]
    coarse = coarse_ref[...]
    jj = jax.lax.broadcasted_iota(jnp.int32, (128, 2048), 0)
    mm = jax.lax.broadcasted_iota(jnp.int32, (128, 2048), 1)
    T = (jj == 16 * (mm // 256) + mm % 16).astype(jnp.float32)
    c16 = jax.lax.broadcasted_iota(jnp.int32, (16, 2048), 0)
    m16 = jax.lax.broadcasted_iota(jnp.int32, (16, 2048), 1)
    M16 = (c16 == (m16 // 16) % 16).astype(jnp.float32)

    M16h = (c16 == (m16 // 16) % 16).astype(jnp.bfloat16)

    updbig = []
    for k in range(_K):
        updk = upd_ref[:, 128 * k:128 * (k + 1)]
        updbig.append(jnp.dot(updk, T, preferred_element_type=jnp.float32))
    iota_n = jax.lax.broadcasted_iota(jnp.int32, (bt, 256), 1)
    sstack = jnp.concatenate(
        [(iota_n == idx_ref[:, k:k + 1]).astype(jnp.bfloat16)
         for k in range(_K)], axis=0)

    pieces = [noop_ref[...]]
    for ch in range(16):
        seg = jnp.dot(coarse[:, 16 * ch:16 * (ch + 1)], M16,
                      preferred_element_type=jnp.float32) - _LOG_F
        sexp = jnp.dot(sstack[:, 16 * ch:16 * (ch + 1)], M16h,
                       preferred_element_type=jnp.float32)
        for k in range(_K):
            seg = seg + sexp[k * bt:(k + 1) * bt] * updbig[k]
        pieces.append(seg)
    out_ref[...] = jnp.concatenate(pieces, axis=-1)


def _full(w):
    return pl.BlockSpec(w.shape, lambda i: (0,) * w.ndim)


def kernel(x, ln1_g, ln1_b, noop_W, noop_b, cW1, cb1, cW2, cb2, cW3, cb3,
           emb, ln2_g, ln2_b, fW1, fb1, fW2, fb2, fW3, fb3):
    B, C = x.shape
    NTOT = cW3.shape[1]
    FTOT = fW3.shape[1]
    f32 = jnp.float32

    def row(v):
        return v.reshape(1, -1)

    bf16 = jnp.bfloat16
    fW1a = fW1[:C].astype(bf16)
    fW1b = fW1[C:].astype(bf16)
    fW2h = fW2.astype(bf16)
    fW3h = fW3.astype(bf16)
    embh = emb.astype(bf16)

    bt1 = 256
    ins1 = (x, row(ln1_g), row(ln1_b), row(noop_W), row(noop_b),
            cW1, row(cb1), cW2, row(cb2), cW3, row(cb3), fW1a)
    noop, t, coarse, idx = pl.pallas_call(
        _coarse_kernel,
        grid=(B // bt1,),
        in_specs=[pl.BlockSpec((bt1, C), lambda i: (i, 0))]
        + [_full(v) for v in ins1[1:]],
        out_specs=[
            pl.BlockSpec((bt1, 1), lambda i: (i, 0)),
            pl.BlockSpec((bt1, C), lambda i: (i, 0)),
            pl.BlockSpec((bt1, NTOT), lambda i: (i, 0)),
            pl.BlockSpec((bt1, _K), lambda i: (i, 0)),
        ],
        out_shape=[
            jax.ShapeDtypeStruct((B, 1), f32),
            jax.ShapeDtypeStruct((B, C), f32),
            jax.ShapeDtypeStruct((B, NTOT), f32),
            jax.ShapeDtypeStruct((B, _K), jnp.int32),
        ],
    )(*ins1)

    bt2 = 256
    ins2 = (t, idx, embh, row(ln2_g), row(ln2_b),
            fW1b, row(fb1), fW2h, row(fb2), fW3h, row(fb3))
    upd = pl.pallas_call(
        _fine_kernel,
        grid=(B // bt2,),
        in_specs=[
            pl.BlockSpec((bt2, C), lambda i: (i, 0)),
            pl.BlockSpec((bt2, _K), lambda i: (i, 0)),
        ] + [_full(v) for v in ins2[2:]],
        out_specs=pl.BlockSpec((bt2, _K * FTOT), lambda i: (i, 0)),
        out_shape=jax.ShapeDtypeStruct((B, _K * FTOT), f32),
    )(*ins2)

    bt3 = 64
    out = pl.pallas_call(
        _expand_kernel,
        grid=(B // bt3,),
        in_specs=[
            pl.BlockSpec((bt3, 1), lambda i: (i, 0)),
            pl.BlockSpec((bt3, NTOT), lambda i: (i, 0)),
            pl.BlockSpec((bt3, _K), lambda i: (i, 0)),
            pl.BlockSpec((bt3, _K * FTOT), lambda i: (i, 0)),
        ],
        out_specs=pl.BlockSpec((bt3, 1 + NTOT * FTOT), lambda i: (i, 0)),
        out_shape=jax.ShapeDtypeStruct((B, 1 + NTOT * FTOT), f32),
    )(noop, coarse, idx, upd)
    return out


# speedup vs baseline: 1.2274x; 1.0220x over previous
import math

import jax
import jax.numpy as jnp
from jax.experimental import pallas as pl
from jax.experimental.pallas import tpu as pltpu

_K = 4
_LOG_F = math.log(128.0)


def _ln_rows(x, g, b, eps=1e-5):
    m = jnp.mean(x, axis=-1, keepdims=True)
    v = jnp.mean((x - m) ** 2, axis=-1, keepdims=True)
    return (x - m) * jax.lax.rsqrt(v + eps) * g + b


def _gelu(x):
    return 0.5 * x * (1.0 + jax.lax.erf(x * (1.0 / math.sqrt(2.0))))


def _coarse_kernel(x_ref, ln1g_ref, ln1b_ref, noopw_ref, noopb_ref,
                   cw1_ref, cb1_ref, cw2_ref, cb2_ref, cw3_ref, cb3_ref,
                   fw1a_ref,
                   noop_ref, t_ref, coarse_ref, idx_ref,
                   fw1a16_ref):
    f32 = jnp.float32

    @pl.when(pl.program_id(0) == 0)
    def _cast_weights():
        fw1a16_ref[...] = fw1a_ref[...].astype(jnp.bfloat16)

    x = x_ref[...]
    xln = _ln_rows(x, ln1g_ref[...], ln1b_ref[...])
    noop_ref[...] = (jnp.dot(xln, noopw_ref[...], preferred_element_type=f32)
                     + noopb_ref[...])
    h = _gelu(jnp.dot(xln, cw1_ref[...], preferred_element_type=f32)
              + cb1_ref[...])
    h = _gelu(jnp.dot(h, cw2_ref[...], preferred_element_type=f32)
              + cb2_ref[...])
    coarse = (jnp.dot(h, cw3_ref[...], preferred_element_type=f32)
              + cb3_ref[...])
    coarse_ref[...] = coarse
    t_ref[...] = jnp.dot(xln.astype(jnp.bfloat16), fw1a16_ref[...],
                         preferred_element_type=f32)

    n = coarse.shape[-1]
    iota = jax.lax.broadcasted_iota(jnp.int32, coarse.shape, 1)
    vals = coarse
    for k in range(_K):
        m = jnp.max(vals, axis=-1, keepdims=True)
        idxk = jnp.min(jnp.where(vals == m, iota, n), axis=-1, keepdims=True)
        idx_ref[:, k:k + 1] = idxk
        vals = jnp.where(iota == idxk, jnp.float32(-jnp.inf), vals)


def _fine_kernel(t_ref, idx_ref, emb_ref, ln2g_ref, ln2b_ref,
                 fw1b_ref, fb1_ref, fw2_ref, fb2_ref, fw3_ref, fb3_ref,
                 upd_ref,
                 emb16_ref, fw1b16_ref, fw2h_ref, fw3h_ref):
    f32 = jnp.float32
    bf16 = jnp.bfloat16

    @pl.when(pl.program_id(0) == 0)
    def _cast_weights():
        emb16_ref[...] = emb_ref[...].astype(bf16)
        fw1b16_ref[...] = fw1b_ref[...].astype(bf16)
        fw2h_ref[...] = fw2_ref[...].astype(bf16)
        fw3h_ref[...] = fw3_ref[...].astype(bf16)

    t = t_ref[...]
    bt = t.shape[0]
    ntot = emb_ref.shape[0]
    ftot = fw3_ref.shape[1]
    iota_n = jax.lax.broadcasted_iota(jnp.int32, (bt, ntot), 1)
    for k in range(_K):
        idxk = idx_ref[:, k:k + 1]
        oh = (iota_n == idxk).astype(bf16)
        e = jnp.dot(oh, emb16_ref[...], preferred_element_type=f32)
        e = _ln_rows(e, ln2g_ref[...], ln2b_ref[...])
        h = _gelu(t + jnp.dot(e.astype(bf16), fw1b16_ref[...],
                              preferred_element_type=f32)
                  + fb1_ref[...])
        h = _gelu(jnp.dot(h.astype(bf16), fw2h_ref[...],
                          preferred_element_type=f32)
                  + fb2_ref[...])
        f = (jnp.dot(h.astype(bf16), fw3h_ref[...],
                     preferred_element_type=f32)
             + fb3_ref[...])
        m = jnp.max(f, axis=-1, keepdims=True)
        lse = m + jnp.log(jnp.sum(jnp.exp(f - m), axis=-1, keepdims=True))
        upd_ref[:, k * ftot:(k + 1) * ftot] = f + _LOG_F - lse


def _expand_kernel(noop_ref, coarse_ref, idx_ref, upd_ref, out_ref):
    bt = coarse_ref.shape[0]
    coarse = coarse_ref[...]
    jj = jax.lax.broadcasted_iota(jnp.int32, (128, 2048), 0)
    mm = jax.lax.broadcasted_iota(jnp.int32, (128, 2048), 1)
    T = (jj == 16 * (mm // 256) + mm % 16).astype(jnp.float32)
    c16 = jax.lax.broadcasted_iota(jnp.int32, (16, 2048), 0)
    m16 = jax.lax.broadcasted_iota(jnp.int32, (16, 2048), 1)
    M16 = (c16 == (m16 // 16) % 16).astype(jnp.float32)
    M16h = (c16 == (m16 // 16) % 16).astype(jnp.bfloat16)

    updbig = []
    for k in range(_K):
        updk = upd_ref[:, 128 * k:128 * (k + 1)]
        updbig.append(jnp.dot(updk, T, preferred_element_type=jnp.float32))
    iota_n = jax.lax.broadcasted_iota(jnp.int32, (bt, 256), 1)
    sstack = jnp.concatenate(
        [(iota_n == idx_ref[:, k:k + 1]).astype(jnp.bfloat16)
         for k in range(_K)], axis=0)

    pieces = [noop_ref[...]]
    for ch in range(16):
        seg = jnp.dot(coarse[:, 16 * ch:16 * (ch + 1)], M16,
                      preferred_element_type=jnp.float32) - _LOG_F
        sexp = jnp.dot(sstack[:, 16 * ch:16 * (ch + 1)], M16h,
                       preferred_element_type=jnp.float32)
        for k in range(_K):
            seg = seg + sexp[k * bt:(k + 1) * bt] * updbig[k]
        pieces.append(seg)
    out_ref[...] = jnp.concatenate(pieces, axis=-1)


def _full(w):
    return pl.BlockSpec(w.shape, lambda i: (0,) * w.ndim)


def kernel(x, ln1_g, ln1_b, noop_W, noop_b, cW1, cb1, cW2, cb2, cW3, cb3,
           emb, ln2_g, ln2_b, fW1, fb1, fW2, fb2, fW3, fb3):
    B, C = x.shape
    NTOT = cW3.shape[1]
    FTOT = fW3.shape[1]
    f32 = jnp.float32
    bf16 = jnp.bfloat16

    bt1 = 256
    ins1 = (x, ln1_g, ln1_b, noop_W, noop_b,
            cW1, cb1, cW2, cb2, cW3, cb3, fW1)
    noop, t, coarse, idx = pl.pallas_call(
        _coarse_kernel,
        grid=(B // bt1,),
        in_specs=[pl.BlockSpec((bt1, C), lambda i: (i, 0))]
        + [_full(v) for v in ins1[1:-1]]
        + [pl.BlockSpec((C, C), lambda i: (0, 0))],
        out_specs=[
            pl.BlockSpec((bt1, 1), lambda i: (i, 0)),
            pl.BlockSpec((bt1, C), lambda i: (i, 0)),
            pl.BlockSpec((bt1, NTOT), lambda i: (i, 0)),
            pl.BlockSpec((bt1, _K), lambda i: (i, 0)),
        ],
        out_shape=[
            jax.ShapeDtypeStruct((B, 1), f32),
            jax.ShapeDtypeStruct((B, C), f32),
            jax.ShapeDtypeStruct((B, NTOT), f32),
            jax.ShapeDtypeStruct((B, _K), jnp.int32),
        ],
        scratch_shapes=[pltpu.VMEM((C, C), bf16)],
    )(*ins1)

    bt2 = 256
    ins2 = (t, idx, emb, ln2_g, ln2_b, fW1, fb1, fW2, fb2, fW3, fb3)
    upd = pl.pallas_call(
        _fine_kernel,
        grid=(B // bt2,),
        in_specs=[
            pl.BlockSpec((bt2, C), lambda i: (i, 0)),
            pl.BlockSpec((bt2, _K), lambda i: (i, 0)),
            _full(emb), _full(ln2_g), _full(ln2_b),
            pl.BlockSpec((C, C), lambda i: (1, 0)),
            _full(fb1), _full(fW2), _full(fb2), _full(fW3), _full(fb3),
        ],
        out_specs=pl.BlockSpec((bt2, _K * FTOT), lambda i: (i, 0)),
        out_shape=jax.ShapeDtypeStruct((B, _K * FTOT), f32),
        scratch_shapes=[
            pltpu.VMEM((NTOT, C), bf16),
            pltpu.VMEM((C, C), bf16),
            pltpu.VMEM((C, C), bf16),
            pltpu.VMEM((C, FTOT), bf16),
        ],
    )(*ins2)

    bt3 = 64
    out = pl.pallas_call(
        _expand_kernel,
        grid=(B // bt3,),
        in_specs=[
            pl.BlockSpec((bt3, 1), lambda i: (i, 0)),
            pl.BlockSpec((bt3, NTOT), lambda i: (i, 0)),
            pl.BlockSpec((bt3, _K), lambda i: (i, 0)),
            pl.BlockSpec((bt3, _K * FTOT), lambda i: (i, 0)),
        ],
        out_specs=pl.BlockSpec((bt3, 1 + NTOT * FTOT), lambda i: (i, 0)),
        out_shape=jax.ShapeDtypeStruct((B, 1 + NTOT * FTOT), f32),
    )(noop, coarse, idx, upd)
    return out


# speedup vs baseline: 1.2534x; 1.0212x over previous
import math

import jax
import jax.numpy as jnp
from jax.experimental import pallas as pl
from jax.experimental.pallas import tpu as pltpu

_K = 4
_LOG_F = math.log(128.0)


def _ln_rows(x, g, b, eps=1e-5):
    m = jnp.mean(x, axis=-1, keepdims=True)
    v = jnp.mean((x - m) ** 2, axis=-1, keepdims=True)
    return (x - m) * jax.lax.rsqrt(v + eps) * g + b


def _gelu(x):
    return 0.5 * x * (1.0 + jax.lax.erf(x * (1.0 / math.sqrt(2.0))))


def _fused_kernel(x_ref, ln1g_ref, ln1b_ref, noopw_ref, noopb_ref,
                  cw1_ref, cb1_ref, cw2_ref, cb2_ref, cw3_ref, cb3_ref,
                  emb_ref, ln2g_ref, ln2b_ref,
                  fw1_ref, fb1_ref, fw2_ref, fb2_ref, fw3_ref, fb3_ref,
                  out_ref,
                  fw1a16_ref, fw1b16_ref, emb16_ref, fw2h_ref, fw3h_ref,
                  noop_s, coarse_s, idx_s, upd_s):
    f32 = jnp.float32
    bf16 = jnp.bfloat16
    i = pl.program_id(0)
    j = pl.program_id(1)
    C = cw1_ref.shape[0]
    ntot = cw3_ref.shape[1]
    ftot = fw3_ref.shape[1]

    @pl.when((i == 0) & (j == 0))
    def _cast_weights():
        fw1a16_ref[...] = fw1_ref[0:C, :].astype(bf16)
        fw1b16_ref[...] = fw1_ref[C:2 * C, :].astype(bf16)
        emb16_ref[...] = emb_ref[...].astype(bf16)
        fw2h_ref[...] = fw2_ref[...].astype(bf16)
        fw3h_ref[...] = fw3_ref[...].astype(bf16)

    @pl.when(j == 0)
    def _coarse_and_fine():
        x = x_ref[...]
        bt = x.shape[0]
        xln = _ln_rows(x, ln1g_ref[...], ln1b_ref[...])
        noop_s[...] = (jnp.dot(xln, noopw_ref[...],
                               preferred_element_type=f32) + noopb_ref[...])
        h = _gelu(jnp.dot(xln, cw1_ref[...], preferred_element_type=f32)
                  + cb1_ref[...])
        h = _gelu(jnp.dot(h, cw2_ref[...], preferred_element_type=f32)
                  + cb2_ref[...])
        coarse = (jnp.dot(h, cw3_ref[...], preferred_element_type=f32)
                  + cb3_ref[...])
        coarse_s[...] = coarse

        iota_n = jax.lax.broadcasted_iota(jnp.int32, (bt, ntot), 1)
        vals = coarse
        idxs = []
        for k in range(_K):
            m = jnp.max(vals, axis=-1, keepdims=True)
            idxk = jnp.min(jnp.where(vals == m, iota_n, ntot), axis=-1,
                           keepdims=True)
            idxs.append(idxk)
            idx_s[:, k:k + 1] = idxk
            vals = jnp.where(iota_n == idxk, jnp.float32(-jnp.inf), vals)

        t = jnp.dot(xln.astype(bf16), fw1a16_ref[...],
                    preferred_element_type=f32)
        for k in range(_K):
            oh = (iota_n == idxs[k]).astype(bf16)
            e = jnp.dot(oh, emb16_ref[...], preferred_element_type=f32)
            e = _ln_rows(e, ln2g_ref[...], ln2b_ref[...])
            hf = _gelu(t + jnp.dot(e.astype(bf16), fw1b16_ref[...],
                                   preferred_element_type=f32)
                       + fb1_ref[...])
            hf = _gelu(jnp.dot(hf.astype(bf16), fw2h_ref[...],
                               preferred_element_type=f32)
                       + fb2_ref[...])
            f = (jnp.dot(hf.astype(bf16), fw3h_ref[...],
                         preferred_element_type=f32)
                 + fb3_ref[...])
            m = jnp.max(f, axis=-1, keepdims=True)
            lse = m + jnp.log(jnp.sum(jnp.exp(f - m), axis=-1,
                                      keepdims=True))
            upd_s[:, k * ftot:(k + 1) * ftot] = f + _LOG_F - lse

    bt = out_ref.shape[0]
    rows = pl.ds(j * bt, bt)
    coarse = coarse_s[rows, :]
    upd = upd_s[rows, :]
    noop = noop_s[rows, :]
    jj = jax.lax.broadcasted_iota(jnp.int32, (ftot, 2048), 0)
    mm = jax.lax.broadcasted_iota(jnp.int32, (ftot, 2048), 1)
    T = (jj == 16 * (mm // 256) + mm % 16).astype(f32)
    c16 = jax.lax.broadcasted_iota(jnp.int32, (16, 2048), 0)
    m16 = jax.lax.broadcasted_iota(jnp.int32, (16, 2048), 1)
    M16 = (c16 == (m16 // 16) % 16).astype(f32)
    M16h = (c16 == (m16 // 16) % 16).astype(bf16)

    updbig = [jnp.dot(upd[:, 128 * k:128 * (k + 1)], T,
                      preferred_element_type=f32) for k in range(_K)]
    iota_e = jax.lax.broadcasted_iota(jnp.int32, (bt, ntot), 1)
    idx_e = idx_s[rows, :]
    sstack = jnp.concatenate(
        [(iota_e == idx_e[:, k:k + 1]).astype(bf16) for k in range(_K)],
        axis=0)

    pieces = [noop]
    for ch in range(16):
        seg = jnp.dot(coarse[:, 16 * ch:16 * (ch + 1)], M16,
                      preferred_element_type=f32) - _LOG_F
        sexp = jnp.dot(sstack[:, 16 * ch:16 * (ch + 1)], M16h,
                       preferred_element_type=f32)
        for k in range(_K):
            seg = seg + sexp[k * bt:(k + 1) * bt] * updbig[k]
        pieces.append(seg)
    out_ref[...] = jnp.concatenate(pieces, axis=-1)


def _full(w):
    return pl.BlockSpec(w.shape, lambda i, j: (0,) * w.ndim)


def kernel(x, ln1_g, ln1_b, noop_W, noop_b, cW1, cb1, cW2, cb2, cW3, cb3,
           emb, ln2_g, ln2_b, fW1, fb1, fW2, fb2, fW3, fb3):
    B, C = x.shape
    NTOT = cW3.shape[1]
    FTOT = fW3.shape[1]
    f32 = jnp.float32
    bf16 = jnp.bfloat16

    BT = 256
    NJ = 4
    bte = BT // NJ

    ins = (x, ln1_g, ln1_b, noop_W, noop_b,
           cW1, cb1, cW2, cb2, cW3, cb3,
           emb, ln2_g, ln2_b, fW1, fb1, fW2, fb2, fW3, fb3)
    out = pl.pallas_call(
        _fused_kernel,
        grid=(B // BT, NJ),
        in_specs=[pl.BlockSpec((BT, C), lambda i, j: (i, 0))]
        + [_full(v) for v in ins[1:]],
        out_specs=pl.BlockSpec((bte, 1 + NTOT * FTOT),
                               lambda i, j: (i * NJ + j, 0)),
        out_shape=jax.ShapeDtypeStruct((B, 1 + NTOT * FTOT), f32),
        scratch_shapes=[
            pltpu.VMEM((C, C), bf16),
            pltpu.VMEM((C, C), bf16),
            pltpu.VMEM((NTOT, C), bf16),
            pltpu.VMEM((C, C), bf16),
            pltpu.VMEM((C, FTOT), bf16),
            pltpu.VMEM((BT, 1), f32),
            pltpu.VMEM((BT, NTOT), f32),
            pltpu.VMEM((BT, _K), jnp.int32),
            pltpu.VMEM((BT, _K * FTOT), f32),
        ],
    )(*ins)
    return out
